# baseline (speedup 1.0000x reference)
import functools

import jax
import jax.numpy as jnp
import numpy as np
from jax import lax
from jax.experimental import pallas as pl
from jax.experimental.pallas import tpu as pltpu
from jax.experimental.pallas import tpu_sc as plsc

BQ, VQ, LQ, CQ, NQ = 8, 4, 256, 256, 5000
NMAPS = BQ * VQ
NS = 2512
NP = NS
TB = 512
NTC = NQ - NS
NTCP = 2560
G = 16
NCHUNKS = NP // G
H = 16
SCALE = H * 1.0 / 224.0

_mesh = plsc.VectorSubcoreMesh(
    core_axis_name="c", subcore_axis_name="s", num_cores=2, num_subcores=16
)


def _rnd16(u):
    return (u + jnp.uint32(0x7FFF) + ((u >> 16) & jnp.uint32(1))) >> 16


def _box_math(bx1, by1, bx2, by2):
    cx = (bx1 + bx2) * 0.5
    cy = (by1 + by2) * 0.5
    nsx = (bx2 - bx1) * 1.1
    nsy = (by2 - by1) * 1.1
    lox = jnp.maximum(cx - nsx * 0.5, 0.0)
    loy = jnp.maximum(cy - nsy * 0.5, 0.0)
    hix = jnp.minimum(cx + nsx * 0.5, 224.0)
    hiy = jnp.minimum(cy + nsy * 0.5, 224.0)
    hix = jnp.maximum(hix, lox + 1e-6)
    hiy = jnp.maximum(hiy, loy + 1e-6)
    lox = jnp.minimum(lox, 224.0)
    loy = jnp.minimum(loy, 224.0)
    hix = jnp.minimum(hix, 224.0)
    hiy = jnp.minimum(hiy, 224.0)
    x1s = lox * SCALE
    y1s = loy * SCALE
    x2s = hix * SCALE
    y2s = hiy * SCALE
    roi_w = jnp.maximum(x2s - x1s, 1.0)
    roi_h = jnp.maximum(y2s - y1s, 1.0)
    sx = x1s + 0.5 * roi_w
    sy = y1s + 0.5 * roi_h
    sx = jnp.minimum(jnp.maximum(sx, 0.0), H - 1.0)
    sy = jnp.minimum(jnp.maximum(sy, 0.0), H - 1.0)
    x0 = jnp.minimum(sx.astype(jnp.int32), H - 2)
    y0 = jnp.minimum(sy.astype(jnp.int32), H - 2)
    lx = sx - x0.astype(jnp.float32)
    ly = sy - y0.astype(jnp.float32)
    return x0, y0, lx, ly, 1.0 - lx, 1.0 - ly


def _body(feats_hbm, boxes_hbm, out_hbm, map_v, box_v, out_v):
    wid = lax.axis_index("s") * 2 + lax.axis_index("c")
    pltpu.sync_copy(feats_hbm.at[wid], map_v)
    pltpu.sync_copy(boxes_hbm.at[wid], box_v)

    def chunk(k, carry):
        g16 = pl.ds(k * G, G)
        x0, y0, lx, ly, hx, hy = _box_math(
            box_v[0, g16], box_v[1, g16], box_v[2, g16], box_v[3, g16]
        )
        o00v = (y0 * H + x0) * (CQ // 2)
        w00r = _rnd16(plsc.bitcast(hy * hx, jnp.uint32))
        w01r = _rnd16(plsc.bitcast(hy * lx, jnp.uint32))
        w10r = _rnd16(plsc.bitcast(ly * hx, jnp.uint32))
        w11r = _rnd16(plsc.bitcast(ly * lx, jnp.uint32))
        w00r = w00r | (w00r << 16)
        w01r = w01r | (w01r << 16)
        w10r = w10r | (w10r << 16)
        w11r = w11r | (w11r << 16)

        for i in range(G):
            o00 = o00v[i]

            def bsplat(wr):
                return plsc.bitcast(
                    jnp.full((16,), wr[i], jnp.uint32), jnp.bfloat16
                )

            w00 = bsplat(w00r)
            w01 = bsplat(w01r)
            w10 = bsplat(w10r)
            w11 = bsplat(w11r)
            accs = []
            CW = CQ // 2
            for j in range(CQ // 32):
                a = map_v[pl.ds(o00 + j * 16, 16)]
                b = map_v[pl.ds(o00 + CW + j * 16, 16)]
                c = map_v[pl.ds(o00 + H * CW + j * 16, 16)]
                d = map_v[pl.ds(o00 + (H + 1) * CW + j * 16, 16)]
                ab = plsc.bitcast(a, jnp.bfloat16)
                bb = plsc.bitcast(b, jnp.bfloat16)
                cb = plsc.bitcast(c, jnp.bfloat16)
                db = plsc.bitcast(d, jnp.bfloat16)
                acc = (w00 * ab + w01 * bb) + (w10 * cb + w11 * db)
                u = plsc.bitcast(acc, jnp.uint32)
                lo = plsc.bitcast(u << 16, jnp.float32)
                hi = plsc.bitcast(u & jnp.uint32(0xFFFF0000), jnp.float32)
                accs.append((lo, hi))
            for j in range(CQ // 32):
                lo, hi = accs[j]
                out_v[i, pl.ds(j * 32, 16)] = lo
                out_v[i, pl.ds(j * 32 + 16, 16)] = hi

        pltpu.sync_copy(out_v, out_hbm.at[wid, pl.ds(k * G, G)])
        return 0

    lax.fori_loop(0, NCHUNKS, chunk, 0)


_sc_call = pl.kernel(
    _body,
    out_type=jax.ShapeDtypeStruct((NMAPS, NP, CQ), jnp.float32),
    mesh=_mesh,
    scratch_types=[
        pltpu.VMEM((LQ * CQ // 2,), jnp.uint32),
        pltpu.VMEM((4, NP), jnp.float32),
        pltpu.VMEM((G, CQ), jnp.float32),
    ],
    compiler_params=pltpu.CompilerParams(needs_layout_passes=False),
)


def _tc_body(boxes_ref, map_ref, out_ref):
    x0, y0, lx, ly, hx, hy = _box_math(
        boxes_ref[0, 0, :], boxes_ref[0, 1, :],
        boxes_ref[0, 2, :], boxes_ref[0, 3, :],
    )
    col = jax.lax.broadcasted_iota(jnp.int32, (TB, H), 1)
    wy = jnp.where(col == y0[:, None], hy[:, None], 0.0) + jnp.where(
        col == y0[:, None] + 1, ly[:, None], 0.0
    )
    wx = jnp.where(col == x0[:, None], hx[:, None], 0.0) + jnp.where(
        col == x0[:, None] + 1, lx[:, None], 0.0
    )
    w = (wy[:, :, None] * wx[:, None, :]).reshape(TB, H * H)
    out_ref[0] = jax.lax.dot_general(
        w.astype(jnp.bfloat16),
        map_ref[0],
        (((1,), (0,)), ((), ())),
        preferred_element_type=jnp.float32,
    )


_tc_call = pl.pallas_call(
    _tc_body,
    grid=(NMAPS, NTCP // TB),
    in_specs=[
        pl.BlockSpec((1, 4, TB), lambda m, t: (m, 0, t)),
        pl.BlockSpec((1, LQ, CQ), lambda m, t: (m, 0, 0)),
    ],
    out_specs=pl.BlockSpec((1, TB, CQ), lambda m, t: (m, t, 0)),
    out_shape=jax.ShapeDtypeStruct((NMAPS, NTCP, CQ), jnp.float32),
)


@jax.jit
def kernel(img_feats, bboxes):
    u = jax.lax.bitcast_convert_type(
        img_feats.reshape(NMAPS, LQ, CQ // 32, 32), jnp.uint32
    )
    r = (u + jnp.uint32(0x7FFF) + ((u >> 16) & jnp.uint32(1))) >> 16
    feats = (r[..., :16] | (r[..., 16:] << 16)).reshape(NMAPS, LQ * CQ // 2)
    boxes = bboxes.reshape(NMAPS, NQ, 4)
    boxes_sc = boxes[:, :NS].transpose(0, 2, 1)
    boxes_tc = jnp.concatenate(
        [boxes[:, NS:], jnp.zeros((NMAPS, NTCP - NTC, 4), jnp.float32)],
        axis=1,
    ).transpose(0, 2, 1)
    feats_bf = img_feats.reshape(NMAPS, LQ, CQ).astype(jnp.bfloat16)
    out_sc = _sc_call(feats, boxes_sc)
    out_tc = _tc_call(boxes_tc, feats_bf)
    out = jnp.concatenate([out_sc, out_tc[:, :NTC]], axis=1)
    return out.reshape(BQ, VQ, NQ, CQ)


# speedup vs baseline: 1.1721x; 1.1721x over previous
import functools

import jax
import jax.numpy as jnp
import numpy as np
from jax import lax
from jax.experimental import pallas as pl
from jax.experimental.pallas import tpu as pltpu
from jax.experimental.pallas import tpu_sc as plsc

BQ, VQ, LQ, CQ, NQ = 8, 4, 256, 256, 5000
NMAPS = BQ * VQ
NS = 2512
NP = NS
TB = 512
NTC = NQ - NS
NTCP = 2560
G = 16
NCHUNKS = NP // G
H = 16
SCALE = H * 1.0 / 224.0

_mesh = plsc.VectorSubcoreMesh(
    core_axis_name="c", subcore_axis_name="s", num_cores=2, num_subcores=16
)


def _rnd16(u):
    return (u + jnp.uint32(0x7FFF) + ((u >> 16) & jnp.uint32(1))) >> 16


def _box_math(bx1, by1, bx2, by2):
    cx = (bx1 + bx2) * 0.5
    cy = (by1 + by2) * 0.5
    nsx = (bx2 - bx1) * 1.1
    nsy = (by2 - by1) * 1.1
    lox = jnp.maximum(cx - nsx * 0.5, 0.0)
    loy = jnp.maximum(cy - nsy * 0.5, 0.0)
    hix = jnp.minimum(cx + nsx * 0.5, 224.0)
    hiy = jnp.minimum(cy + nsy * 0.5, 224.0)
    hix = jnp.maximum(hix, lox + 1e-6)
    hiy = jnp.maximum(hiy, loy + 1e-6)
    lox = jnp.minimum(lox, 224.0)
    loy = jnp.minimum(loy, 224.0)
    hix = jnp.minimum(hix, 224.0)
    hiy = jnp.minimum(hiy, 224.0)
    x1s = lox * SCALE
    y1s = loy * SCALE
    x2s = hix * SCALE
    y2s = hiy * SCALE
    roi_w = jnp.maximum(x2s - x1s, 1.0)
    roi_h = jnp.maximum(y2s - y1s, 1.0)
    sx = x1s + 0.5 * roi_w
    sy = y1s + 0.5 * roi_h
    sx = jnp.minimum(jnp.maximum(sx, 0.0), H - 1.0)
    sy = jnp.minimum(jnp.maximum(sy, 0.0), H - 1.0)
    x0 = jnp.minimum(sx.astype(jnp.int32), H - 2)
    y0 = jnp.minimum(sy.astype(jnp.int32), H - 2)
    lx = sx - x0.astype(jnp.float32)
    ly = sy - y0.astype(jnp.float32)
    return x0, y0, lx, ly, 1.0 - lx, 1.0 - ly


def _body(feats_hbm, boxes_hbm, out_hbm, map_v, box_v, out_v):
    wid = lax.axis_index("s") * 2 + lax.axis_index("c")
    pltpu.sync_copy(feats_hbm.at[wid], map_v)
    pltpu.sync_copy(boxes_hbm.at[wid], box_v)

    def chunk(k, carry):
        g16 = pl.ds(k * G, G)
        x0, y0, lx, ly, hx, hy = _box_math(
            box_v[0, g16], box_v[1, g16], box_v[2, g16], box_v[3, g16]
        )
        o00v = (y0 * H + x0) * (CQ // 2)
        w00r = _rnd16(plsc.bitcast(hy * hx, jnp.uint32))
        w01r = _rnd16(plsc.bitcast(hy * lx, jnp.uint32))
        w10r = _rnd16(plsc.bitcast(ly * hx, jnp.uint32))
        w11r = _rnd16(plsc.bitcast(ly * lx, jnp.uint32))
        w00r = w00r | (w00r << 16)
        w01r = w01r | (w01r << 16)
        w10r = w10r | (w10r << 16)
        w11r = w11r | (w11r << 16)

        for i in range(G):
            o00 = o00v[i]

            def bsplat(wr):
                return plsc.bitcast(
                    jnp.full((16,), wr[i], jnp.uint32), jnp.bfloat16
                )

            w00 = bsplat(w00r)
            w01 = bsplat(w01r)
            w10 = bsplat(w10r)
            w11 = bsplat(w11r)
            accs = []
            CW = CQ // 2
            for j in range(CQ // 32):
                a = map_v[pl.ds(o00 + j * 16, 16)]
                b = map_v[pl.ds(o00 + CW + j * 16, 16)]
                c = map_v[pl.ds(o00 + H * CW + j * 16, 16)]
                d = map_v[pl.ds(o00 + (H + 1) * CW + j * 16, 16)]
                ab = plsc.bitcast(a, jnp.bfloat16)
                bb = plsc.bitcast(b, jnp.bfloat16)
                cb = plsc.bitcast(c, jnp.bfloat16)
                db = plsc.bitcast(d, jnp.bfloat16)
                acc = (w00 * ab + w01 * bb) + (w10 * cb + w11 * db)
                u = plsc.bitcast(acc, jnp.uint32)
                lo = plsc.bitcast(u << 16, jnp.float32)
                hi = plsc.bitcast(u & jnp.uint32(0xFFFF0000), jnp.float32)
                accs.append((lo, hi))
            for j in range(CQ // 32):
                lo, hi = accs[j]
                out_v[i, pl.ds(j * 32, 16)] = lo
                out_v[i, pl.ds(j * 32 + 16, 16)] = hi

        pltpu.sync_copy(out_v, out_hbm.at[wid, pl.ds(k * G, G)])
        return 0

    lax.fori_loop(0, NCHUNKS, chunk, 0)


_sc_call = pl.kernel(
    _body,
    out_type=jax.ShapeDtypeStruct((NMAPS, NP, CQ), jnp.float32),
    mesh=_mesh,
    scratch_types=[
        pltpu.VMEM((LQ * CQ // 2,), jnp.uint32),
        pltpu.VMEM((4, NP), jnp.float32),
        pltpu.VMEM((G, CQ), jnp.float32),
    ],
    compiler_params=pltpu.CompilerParams(needs_layout_passes=False),
)


def _tc_body(boxes_ref, map_ref, out_ref):
    x0, y0, lx, ly, hx, hy = _box_math(
        boxes_ref[0, :, 0:1], boxes_ref[0, :, 1:2],
        boxes_ref[0, :, 2:3], boxes_ref[0, :, 3:4],
    )
    lcol = jax.lax.broadcasted_iota(jnp.int32, (TB, H * H), 1)
    ycol = lcol >> 4
    xcol = lcol & 15
    wy = jnp.where(ycol == y0, hy, 0.0) + jnp.where(ycol == y0 + 1, ly, 0.0)
    wx = jnp.where(xcol == x0, hx, 0.0) + jnp.where(xcol == x0 + 1, lx, 0.0)
    out_ref[0] = jax.lax.dot_general(
        (wy * wx).astype(jnp.bfloat16),
        map_ref[0],
        (((1,), (0,)), ((), ())),
        preferred_element_type=jnp.float32,
    )


_tc_call = pl.pallas_call(
    _tc_body,
    grid=(NMAPS, NTCP // TB),
    in_specs=[
        pl.BlockSpec((1, TB, 4), lambda m, t: (m, t, 0)),
        pl.BlockSpec((1, LQ, CQ), lambda m, t: (m, 0, 0)),
    ],
    out_specs=pl.BlockSpec((1, TB, CQ), lambda m, t: (m, t, 0)),
    out_shape=jax.ShapeDtypeStruct((NMAPS, NTCP, CQ), jnp.float32),
)


@jax.jit
def kernel(img_feats, bboxes):
    u = jax.lax.bitcast_convert_type(
        img_feats.reshape(NMAPS, LQ, CQ // 32, 32), jnp.uint32
    )
    r = (u + jnp.uint32(0x7FFF) + ((u >> 16) & jnp.uint32(1))) >> 16
    feats = (r[..., :16] | (r[..., 16:] << 16)).reshape(NMAPS, LQ * CQ // 2)
    boxes = bboxes.reshape(NMAPS, NQ, 4)
    boxes_sc = boxes[:, :NS].transpose(0, 2, 1)
    boxes_tc = jnp.concatenate(
        [boxes[:, NS:], jnp.zeros((NMAPS, NTCP - NTC, 4), jnp.float32)],
        axis=1,
    )
    feats_bf = img_feats.reshape(NMAPS, LQ, CQ).astype(jnp.bfloat16)
    out_sc = _sc_call(feats, boxes_sc)
    out_tc = _tc_call(boxes_tc, feats_bf)
    out = jnp.concatenate([out_sc, out_tc[:, :NTC]], axis=1)
    return out.reshape(BQ, VQ, NQ, CQ)


# speedup vs baseline: 1.3615x; 1.1617x over previous
import functools

import jax
import jax.numpy as jnp
import numpy as np
from jax import lax
from jax.experimental import pallas as pl
from jax.experimental.pallas import tpu as pltpu
from jax.experimental.pallas import tpu_sc as plsc

BQ, VQ, LQ, CQ, NQ = 8, 4, 256, 256, 5000
NMAPS = BQ * VQ
H = 16
SCALE = H * 1.0 / 224.0
TB = 512
NTC = 2048
NSC = NQ - NTC
NSCP = 2960
G = 16
NFULL = NSC // G
CW = CQ // 2

_mesh = plsc.VectorSubcoreMesh(
    core_axis_name="c", subcore_axis_name="s", num_cores=2, num_subcores=16
)


def _rnd16(u):
    return (u + jnp.uint32(0x7FFF) + ((u >> 16) & jnp.uint32(1))) >> 16


def _box_math(bx1, by1, bx2, by2):
    cx = (bx1 + bx2) * 0.5
    cy = (by1 + by2) * 0.5
    nsx = (bx2 - bx1) * 1.1
    nsy = (by2 - by1) * 1.1
    lox = jnp.maximum(cx - nsx * 0.5, 0.0)
    loy = jnp.maximum(cy - nsy * 0.5, 0.0)
    hix = jnp.minimum(cx + nsx * 0.5, 224.0)
    hiy = jnp.minimum(cy + nsy * 0.5, 224.0)
    hix = jnp.maximum(hix, lox + 1e-6)
    hiy = jnp.maximum(hiy, loy + 1e-6)
    lox = jnp.minimum(lox, 224.0)
    loy = jnp.minimum(loy, 224.0)
    hix = jnp.minimum(hix, 224.0)
    hiy = jnp.minimum(hiy, 224.0)
    x1s = lox * SCALE
    y1s = loy * SCALE
    x2s = hix * SCALE
    y2s = hiy * SCALE
    roi_w = jnp.maximum(x2s - x1s, 1.0)
    roi_h = jnp.maximum(y2s - y1s, 1.0)
    sx = x1s + 0.5 * roi_w
    sy = y1s + 0.5 * roi_h
    sx = jnp.minimum(jnp.maximum(sx, 0.0), H - 1.0)
    sy = jnp.minimum(jnp.maximum(sy, 0.0), H - 1.0)
    x0 = jnp.minimum(sx.astype(jnp.int32), H - 2)
    y0 = jnp.minimum(sy.astype(jnp.int32), H - 2)
    lx = sx - x0.astype(jnp.float32)
    ly = sy - y0.astype(jnp.float32)
    return x0, y0, lx, ly, 1.0 - lx, 1.0 - ly


def _sc_body(feats_hbm, boxes_hbm, out_hbm, stage_v, map_v, box_v, out_v):
    wid = lax.axis_index("s") * 2 + lax.axis_index("c")
    pltpu.sync_copy(boxes_hbm.at[wid], box_v)

    SW = 8192
    for piece in range(LQ * CQ // SW):
        pltpu.sync_copy(feats_hbm.at[wid, pl.ds(piece * SW, SW)], stage_v)

        def conv(t, carry, piece=piece):
            for u2 in range(4):
                r = t * 4 + u2
                a = stage_v[pl.ds(r * 32, 16)]
                b = stage_v[pl.ds(r * 32 + 16, 16)]
                map_v[pl.ds(piece * (SW // 2) + r * 16, 16)] = _rnd16(
                    plsc.bitcast(a, jnp.uint32)
                ) | (_rnd16(plsc.bitcast(b, jnp.uint32)) << 16)
            return carry

        lax.fori_loop(0, SW // 128, conv, 0)

    def chunk(k, store_g):
        g16 = pl.ds(k * G, G)
        x0, y0, lx, ly, hx, hy = _box_math(
            box_v[0, g16], box_v[1, g16], box_v[2, g16], box_v[3, g16]
        )
        o00v = (y0 * H + x0) * CW
        w00r = _rnd16(plsc.bitcast(hy * hx, jnp.uint32))
        w01r = _rnd16(plsc.bitcast(hy * lx, jnp.uint32))
        w10r = _rnd16(plsc.bitcast(ly * hx, jnp.uint32))
        w11r = _rnd16(plsc.bitcast(ly * lx, jnp.uint32))
        w00r = w00r | (w00r << 16)
        w01r = w01r | (w01r << 16)
        w10r = w10r | (w10r << 16)
        w11r = w11r | (w11r << 16)

        for i in range(G):
            o00 = o00v[i]

            def bsplat(wr):
                return plsc.bitcast(
                    jnp.full((16,), wr[i], jnp.uint32), jnp.bfloat16
                )

            w00 = bsplat(w00r)
            w01 = bsplat(w01r)
            w10 = bsplat(w10r)
            w11 = bsplat(w11r)
            accs = []
            for j in range(CQ // 32):
                a = map_v[pl.ds(o00 + j * 16, 16)]
                b = map_v[pl.ds(o00 + CW + j * 16, 16)]
                c = map_v[pl.ds(o00 + H * CW + j * 16, 16)]
                d = map_v[pl.ds(o00 + (H + 1) * CW + j * 16, 16)]
                ab = plsc.bitcast(a, jnp.bfloat16)
                bb = plsc.bitcast(b, jnp.bfloat16)
                cb = plsc.bitcast(c, jnp.bfloat16)
                db = plsc.bitcast(d, jnp.bfloat16)
                acc = (w00 * ab + w01 * bb) + (w10 * cb + w11 * db)
                u = plsc.bitcast(acc, jnp.uint32)
                lo = plsc.bitcast(u << 16, jnp.float32)
                hi = plsc.bitcast(u & jnp.uint32(0xFFFF0000), jnp.float32)
                accs.append((lo, hi))
            for j in range(CQ // 32):
                lo, hi = accs[j]
                out_v[i, pl.ds(j * 32, 16)] = lo
                out_v[i, pl.ds(j * 32 + 16, 16)] = hi

        pltpu.sync_copy(
            out_v.at[pl.ds(0, store_g)],
            out_hbm.at[wid, pl.ds(NTC + k * G, store_g)],
        )

    def full_chunk(k, carry):
        chunk(k, G)
        return carry

    lax.fori_loop(0, NFULL, full_chunk, 0)
    chunk(NFULL, NSC - NFULL * G)


_sc_call = pl.kernel(
    _sc_body,
    out_type=jax.ShapeDtypeStruct((NMAPS, NQ, CQ), jnp.float32),
    mesh=_mesh,
    scratch_types=[
        pltpu.VMEM((8192,), jnp.float32),
        pltpu.VMEM((LQ * CQ // 2,), jnp.uint32),
        pltpu.VMEM((4, NSCP), jnp.float32),
        pltpu.VMEM((G, CQ), jnp.float32),
    ],
    compiler_params=pltpu.CompilerParams(needs_layout_passes=False),
)


def _tc_body(boxes_ref, map_ref, alias_ref, out_ref):
    del alias_ref
    x0, y0, lx, ly, hx, hy = _box_math(
        boxes_ref[0, :, 0:1], boxes_ref[0, :, 1:2],
        boxes_ref[0, :, 2:3], boxes_ref[0, :, 3:4],
    )
    lcol = jax.lax.broadcasted_iota(jnp.int32, (TB, H * H), 1)
    ycol = lcol >> 4
    xcol = lcol & 15
    wy = jnp.where(ycol == y0, hy, 0.0) + jnp.where(ycol == y0 + 1, ly, 0.0)
    wx = jnp.where(xcol == x0, hx, 0.0) + jnp.where(xcol == x0 + 1, lx, 0.0)
    out_ref[0] = jax.lax.dot_general(
        (wy * wx).astype(jnp.bfloat16),
        map_ref[0].astype(jnp.bfloat16),
        (((1,), (0,)), ((), ())),
        preferred_element_type=jnp.float32,
    )


_tc_call = pl.pallas_call(
    _tc_body,
    grid=(NMAPS, NTC // TB),
    in_specs=[
        pl.BlockSpec((1, TB, 4), lambda m, t: (m, t, 0)),
        pl.BlockSpec((1, LQ, CQ), lambda m, t: (m, 0, 0)),
        pl.BlockSpec(memory_space=pl.ANY),
    ],
    out_specs=pl.BlockSpec((1, TB, CQ), lambda m, t: (m, t, 0)),
    out_shape=jax.ShapeDtypeStruct((NMAPS, NQ, CQ), jnp.float32),
    input_output_aliases={2: 0},
)


@jax.jit
def kernel(img_feats, bboxes):
    feats_flat = img_feats.reshape(NMAPS, LQ * CQ)
    boxes = bboxes.reshape(NMAPS, NQ, 4)
    boxes_sc = jnp.concatenate(
        [boxes[:, NTC:], jnp.zeros((NMAPS, NSCP - NSC, 4), jnp.float32)],
        axis=1,
    ).transpose(0, 2, 1)
    out_sc = _sc_call(feats_flat, boxes_sc)
    out = _tc_call(
        boxes, img_feats.reshape(NMAPS, LQ, CQ), out_sc
    )
    return out.reshape(BQ, VQ, NQ, CQ)


# speedup vs baseline: 1.7568x; 1.2903x over previous
import functools

import jax
import jax.numpy as jnp
import numpy as np
from jax import lax
from jax.experimental import pallas as pl
from jax.experimental.pallas import tpu as pltpu
from jax.experimental.pallas import tpu_sc as plsc

BQ, VQ, LQ, CQ, NQ = 8, 4, 256, 256, 5000
NMAPS = BQ * VQ
H = 16
SCALE = H * 1.0 / 224.0
TB = 512
NTC = 2048
NSC = NQ - NTC
NSCP = 2960
G = 16
NFULL = NSC // G
CW = CQ // 2

_mesh = plsc.VectorSubcoreMesh(
    core_axis_name="c", subcore_axis_name="s", num_cores=2, num_subcores=16
)


def _rnd16(u):
    return (u + jnp.uint32(0x7FFF) + ((u >> 16) & jnp.uint32(1))) >> 16


def _box_math(bx1, by1, bx2, by2):
    cx = (bx1 + bx2) * 0.5
    cy = (by1 + by2) * 0.5
    nsx = (bx2 - bx1) * 1.1
    nsy = (by2 - by1) * 1.1
    lox = jnp.maximum(cx - nsx * 0.5, 0.0)
    loy = jnp.maximum(cy - nsy * 0.5, 0.0)
    hix = jnp.minimum(cx + nsx * 0.5, 224.0)
    hiy = jnp.minimum(cy + nsy * 0.5, 224.0)
    hix = jnp.maximum(hix, lox + 1e-6)
    hiy = jnp.maximum(hiy, loy + 1e-6)
    lox = jnp.minimum(lox, 224.0)
    loy = jnp.minimum(loy, 224.0)
    hix = jnp.minimum(hix, 224.0)
    hiy = jnp.minimum(hiy, 224.0)
    x1s = lox * SCALE
    y1s = loy * SCALE
    x2s = hix * SCALE
    y2s = hiy * SCALE
    roi_w = jnp.maximum(x2s - x1s, 1.0)
    roi_h = jnp.maximum(y2s - y1s, 1.0)
    sx = x1s + 0.5 * roi_w
    sy = y1s + 0.5 * roi_h
    sx = jnp.minimum(jnp.maximum(sx, 0.0), H - 1.0)
    sy = jnp.minimum(jnp.maximum(sy, 0.0), H - 1.0)
    x0 = jnp.minimum(sx.astype(jnp.int32), H - 2)
    y0 = jnp.minimum(sy.astype(jnp.int32), H - 2)
    lx = sx - x0.astype(jnp.float32)
    ly = sy - y0.astype(jnp.float32)
    return x0, y0, lx, ly, 1.0 - lx, 1.0 - ly


def _sc_body(feats_hbm, boxes_hbm, out_hbm, stage_v, map_v, box_v, out_v):
    wid = lax.axis_index("s") * 2 + lax.axis_index("c")
    pltpu.sync_copy(boxes_hbm.at[wid], box_v)

    SW = 8192
    for piece in range(LQ * CQ // SW):
        pltpu.sync_copy(feats_hbm.at[wid, pl.ds(piece * SW, SW)], stage_v)

        def conv(t, carry, piece=piece):
            for u2 in range(4):
                r = t * 4 + u2
                a = stage_v[pl.ds(r * 32, 16)]
                b = stage_v[pl.ds(r * 32 + 16, 16)]
                map_v[pl.ds(piece * (SW // 2) + r * 16, 16)] = _rnd16(
                    plsc.bitcast(a, jnp.uint32)
                ) | (_rnd16(plsc.bitcast(b, jnp.uint32)) << 16)
            return carry

        lax.fori_loop(0, SW // 128, conv, 0)

    def chunk(k, store_g):
        g16 = pl.ds(k * G, G)
        x0, y0, lx, ly, hx, hy = _box_math(
            box_v[0, g16], box_v[1, g16], box_v[2, g16], box_v[3, g16]
        )
        o00v = (y0 * H + x0) * CW
        w00r = _rnd16(plsc.bitcast(hy * hx, jnp.uint32))
        w01r = _rnd16(plsc.bitcast(hy * lx, jnp.uint32))
        w10r = _rnd16(plsc.bitcast(ly * hx, jnp.uint32))
        w11r = _rnd16(plsc.bitcast(ly * lx, jnp.uint32))
        w00r = w00r | (w00r << 16)
        w01r = w01r | (w01r << 16)
        w10r = w10r | (w10r << 16)
        w11r = w11r | (w11r << 16)

        for i in range(G):
            o00 = o00v[i]

            def bsplat(wr):
                return plsc.bitcast(
                    jnp.full((16,), wr[i], jnp.uint32), jnp.bfloat16
                )

            w00 = bsplat(w00r)
            w01 = bsplat(w01r)
            w10 = bsplat(w10r)
            w11 = bsplat(w11r)
            accs = []
            for j in range(CQ // 32):
                a = map_v[pl.ds(o00 + j * 16, 16)]
                b = map_v[pl.ds(o00 + CW + j * 16, 16)]
                c = map_v[pl.ds(o00 + H * CW + j * 16, 16)]
                d = map_v[pl.ds(o00 + (H + 1) * CW + j * 16, 16)]
                ab = plsc.bitcast(a, jnp.bfloat16)
                bb = plsc.bitcast(b, jnp.bfloat16)
                cb = plsc.bitcast(c, jnp.bfloat16)
                db = plsc.bitcast(d, jnp.bfloat16)
                acc = (w00 * ab + w01 * bb) + (w10 * cb + w11 * db)
                u = plsc.bitcast(acc, jnp.uint32)
                lo = plsc.bitcast(u << 16, jnp.float32)
                hi = plsc.bitcast(u & jnp.uint32(0xFFFF0000), jnp.float32)
                accs.append((lo, hi))
            for j in range(CQ // 32):
                lo, hi = accs[j]
                out_v[i, pl.ds(j * 32, 16)] = lo
                out_v[i, pl.ds(j * 32 + 16, 16)] = hi

        pltpu.sync_copy(
            out_v.at[pl.ds(0, store_g)],
            out_hbm.at[wid, pl.ds(NTC + k * G, store_g)],
        )

    def full_chunk(k, carry):
        chunk(k, G)
        return carry

    lax.fori_loop(0, NFULL, full_chunk, 0)
    chunk(NFULL, NSC - NFULL * G)


_sc_call = pl.kernel(
    _sc_body,
    out_type=jax.ShapeDtypeStruct((NMAPS, NQ, CQ), jnp.float32),
    mesh=_mesh,
    scratch_types=[
        pltpu.VMEM((8192,), jnp.float32),
        pltpu.VMEM((LQ * CQ // 2,), jnp.uint32),
        pltpu.VMEM((4, NSCP), jnp.float32),
        pltpu.VMEM((G, CQ), jnp.float32),
    ],
    compiler_params=pltpu.CompilerParams(needs_layout_passes=False),
)


def _tc_body(boxes_ref, map_ref, out_ref):
    x0, y0, lx, ly, hx, hy = _box_math(
        boxes_ref[0, :, 0:1], boxes_ref[0, :, 1:2],
        boxes_ref[0, :, 2:3], boxes_ref[0, :, 3:4],
    )
    lcol = jax.lax.broadcasted_iota(jnp.int32, (TB, H * H), 1)
    ycol = lcol >> 4
    xcol = lcol & 15
    wy = jnp.where(ycol == y0, hy, 0.0) + jnp.where(ycol == y0 + 1, ly, 0.0)
    wx = jnp.where(xcol == x0, hx, 0.0) + jnp.where(xcol == x0 + 1, lx, 0.0)
    out_ref[0] = jax.lax.dot_general(
        (wy * wx).astype(jnp.bfloat16),
        map_ref[0].astype(jnp.bfloat16),
        (((1,), (0,)), ((), ())),
        preferred_element_type=jnp.float32,
    )


_tc_call = pl.pallas_call(
    _tc_body,
    grid=(NMAPS, NTC // TB),
    in_specs=[
        pl.BlockSpec((1, TB, 4), lambda m, t: (m, t, 0)),
        pl.BlockSpec((1, LQ, CQ), lambda m, t: (m, 0, 0)),
    ],
    out_specs=pl.BlockSpec((1, TB, CQ), lambda m, t: (m, t, 0)),
    out_shape=jax.ShapeDtypeStruct((NMAPS, NTC, CQ), jnp.float32),
)


@jax.jit
def kernel(img_feats, bboxes):
    feats_flat = img_feats.reshape(NMAPS, LQ * CQ)
    boxes = bboxes.reshape(NMAPS, NQ, 4)
    boxes_sc = jnp.concatenate(
        [boxes[:, NTC:], jnp.zeros((NMAPS, NSCP - NSC, 4), jnp.float32)],
        axis=1,
    ).transpose(0, 2, 1)
    out_sc = _sc_call(feats_flat, boxes_sc)
    out_tc = _tc_call(boxes, img_feats.reshape(NMAPS, LQ, CQ))
    out = lax.dynamic_update_slice(out_sc, out_tc, (0, 0, 0))
    return out.reshape(BQ, VQ, NQ, CQ)


# speedup vs baseline: 1.9615x; 1.1165x over previous
import functools

import jax
import jax.numpy as jnp
import numpy as np
from jax import lax
from jax.experimental import pallas as pl
from jax.experimental.pallas import tpu as pltpu
from jax.experimental.pallas import tpu_sc as plsc

BQ, VQ, LQ, CQ, NQ = 8, 4, 256, 256, 5000
NMAPS = BQ * VQ
H = 16
SCALE = H * 1.0 / 224.0
TB = 512
NTC = 2048
NSC = NQ - NTC
NSCP = 2960
BSTG = 11904
G = 16
NFULL = NSC // G
CW = CQ // 2

_mesh = plsc.VectorSubcoreMesh(
    core_axis_name="c", subcore_axis_name="s", num_cores=2, num_subcores=16
)


def _rnd16(u):
    return (u + jnp.uint32(0x7FFF) + ((u >> 16) & jnp.uint32(1))) >> 16


def _box_math(bx1, by1, bx2, by2):
    cx = (bx1 + bx2) * 0.5
    cy = (by1 + by2) * 0.5
    nsx = (bx2 - bx1) * 1.1
    nsy = (by2 - by1) * 1.1
    lox = jnp.maximum(cx - nsx * 0.5, 0.0)
    loy = jnp.maximum(cy - nsy * 0.5, 0.0)
    hix = jnp.minimum(cx + nsx * 0.5, 224.0)
    hiy = jnp.minimum(cy + nsy * 0.5, 224.0)
    hix = jnp.maximum(hix, lox + 1e-6)
    hiy = jnp.maximum(hiy, loy + 1e-6)
    lox = jnp.minimum(lox, 224.0)
    loy = jnp.minimum(loy, 224.0)
    hix = jnp.minimum(hix, 224.0)
    hiy = jnp.minimum(hiy, 224.0)
    x1s = lox * SCALE
    y1s = loy * SCALE
    x2s = hix * SCALE
    y2s = hiy * SCALE
    roi_w = jnp.maximum(x2s - x1s, 1.0)
    roi_h = jnp.maximum(y2s - y1s, 1.0)
    sx = x1s + 0.5 * roi_w
    sy = y1s + 0.5 * roi_h
    sx = jnp.minimum(jnp.maximum(sx, 0.0), H - 1.0)
    sy = jnp.minimum(jnp.maximum(sy, 0.0), H - 1.0)
    x0 = jnp.minimum(sx.astype(jnp.int32), H - 2)
    y0 = jnp.minimum(sy.astype(jnp.int32), H - 2)
    lx = sx - x0.astype(jnp.float32)
    ly = sy - y0.astype(jnp.float32)
    return x0, y0, lx, ly, 1.0 - lx, 1.0 - ly


def _sc_body(feats_hbm, boxes_hbm, out_hbm, stage_v, map_v, box_v, out_v):
    wid = lax.axis_index("s") * 2 + lax.axis_index("c")
    pltpu.sync_copy(boxes_hbm.at[wid, pl.ds(NTC * 4, BSTG)], box_v)

    SW = 8192
    for piece in range(LQ * CQ // SW):
        pltpu.sync_copy(feats_hbm.at[wid, pl.ds(piece * SW, SW)], stage_v)

        def conv(t, carry, piece=piece):
            for u2 in range(4):
                r = t * 4 + u2
                a = stage_v[pl.ds(r * 32, 16)]
                b = stage_v[pl.ds(r * 32 + 16, 16)]
                map_v[pl.ds(piece * (SW // 2) + r * 16, 16)] = _rnd16(
                    plsc.bitcast(a, jnp.uint32)
                ) | (_rnd16(plsc.bitcast(b, jnp.uint32)) << 16)
            return carry

        lax.fori_loop(0, SW // 128, conv, 0)

    def chunk(k, store_g):
        idx4 = lax.iota(jnp.int32, 16) * 4 + k * (G * 4)
        x0, y0, lx, ly, hx, hy = _box_math(
            plsc.load_gather(box_v, [idx4]),
            plsc.load_gather(box_v, [idx4 + 1]),
            plsc.load_gather(box_v, [idx4 + 2]),
            plsc.load_gather(box_v, [idx4 + 3]),
        )
        o00v = (y0 * H + x0) * CW
        w00r = _rnd16(plsc.bitcast(hy * hx, jnp.uint32))
        w01r = _rnd16(plsc.bitcast(hy * lx, jnp.uint32))
        w10r = _rnd16(plsc.bitcast(ly * hx, jnp.uint32))
        w11r = _rnd16(plsc.bitcast(ly * lx, jnp.uint32))
        w00r = w00r | (w00r << 16)
        w01r = w01r | (w01r << 16)
        w10r = w10r | (w10r << 16)
        w11r = w11r | (w11r << 16)

        for i in range(G):
            o00 = o00v[i]

            def bsplat(wr):
                return plsc.bitcast(
                    jnp.full((16,), wr[i], jnp.uint32), jnp.bfloat16
                )

            w00 = bsplat(w00r)
            w01 = bsplat(w01r)
            w10 = bsplat(w10r)
            w11 = bsplat(w11r)
            accs = []
            for j in range(CQ // 32):
                a = map_v[pl.ds(o00 + j * 16, 16)]
                b = map_v[pl.ds(o00 + CW + j * 16, 16)]
                c = map_v[pl.ds(o00 + H * CW + j * 16, 16)]
                d = map_v[pl.ds(o00 + (H + 1) * CW + j * 16, 16)]
                ab = plsc.bitcast(a, jnp.bfloat16)
                bb = plsc.bitcast(b, jnp.bfloat16)
                cb = plsc.bitcast(c, jnp.bfloat16)
                db = plsc.bitcast(d, jnp.bfloat16)
                acc = (w00 * ab + w01 * bb) + (w10 * cb + w11 * db)
                u = plsc.bitcast(acc, jnp.uint32)
                lo = plsc.bitcast(u << 16, jnp.float32)
                hi = plsc.bitcast(u & jnp.uint32(0xFFFF0000), jnp.float32)
                accs.append((lo, hi))
            for j in range(CQ // 32):
                lo, hi = accs[j]
                out_v[i, pl.ds(j * 32, 16)] = lo
                out_v[i, pl.ds(j * 32 + 16, 16)] = hi

        pltpu.sync_copy(
            out_v.at[pl.ds(0, store_g)],
            out_hbm.at[wid, pl.ds(NTC + k * G, store_g)],
        )

    def full_chunk(k, carry):
        chunk(k, G)
        return carry

    lax.fori_loop(0, NFULL, full_chunk, 0)
    chunk(NFULL, NSC - NFULL * G)


_sc_call = pl.kernel(
    _sc_body,
    out_type=jax.ShapeDtypeStruct((NMAPS, NQ, CQ), jnp.float32),
    mesh=_mesh,
    scratch_types=[
        pltpu.VMEM((8192,), jnp.float32),
        pltpu.VMEM((LQ * CQ // 2,), jnp.uint32),
        pltpu.VMEM((BSTG,), jnp.float32),
        pltpu.VMEM((G, CQ), jnp.float32),
    ],
    compiler_params=pltpu.CompilerParams(needs_layout_passes=False),
)


def _tc_body(boxes_ref, map_ref, out_ref):
    x0, y0, lx, ly, hx, hy = _box_math(
        boxes_ref[0, :, 0:1], boxes_ref[0, :, 1:2],
        boxes_ref[0, :, 2:3], boxes_ref[0, :, 3:4],
    )
    lcol = jax.lax.broadcasted_iota(jnp.int32, (TB, H * H), 1)
    ycol = lcol >> 4
    xcol = lcol & 15
    wy = jnp.where(ycol == y0, hy, 0.0) + jnp.where(ycol == y0 + 1, ly, 0.0)
    wx = jnp.where(xcol == x0, hx, 0.0) + jnp.where(xcol == x0 + 1, lx, 0.0)
    out_ref[0] = jax.lax.dot_general(
        (wy * wx).astype(jnp.bfloat16),
        map_ref[0].astype(jnp.bfloat16),
        (((1,), (0,)), ((), ())),
        preferred_element_type=jnp.float32,
    )


_tc_call = pl.pallas_call(
    _tc_body,
    grid=(NMAPS, NTC // TB),
    in_specs=[
        pl.BlockSpec((1, TB, 4), lambda m, t: (m, t, 0)),
        pl.BlockSpec((1, LQ, CQ), lambda m, t: (m, 0, 0)),
    ],
    out_specs=pl.BlockSpec((1, TB, CQ), lambda m, t: (m, t, 0)),
    out_shape=jax.ShapeDtypeStruct((NMAPS, NTC, CQ), jnp.float32),
)


@jax.jit
def kernel(img_feats, bboxes):
    feats_flat = img_feats.reshape(NMAPS, LQ * CQ)
    boxes = bboxes.reshape(NMAPS, NQ, 4)
    boxes_flat = jnp.pad(
        bboxes.reshape(NMAPS, NQ * 4), ((0, 0), (0, NTC * 4 + BSTG - NQ * 4))
    )
    out_sc = _sc_call(feats_flat, boxes_flat)
    out_tc = _tc_call(boxes, img_feats.reshape(NMAPS, LQ, CQ))
    out = lax.dynamic_update_slice(out_sc, out_tc, (0, 0, 0))
    return out.reshape(BQ, VQ, NQ, CQ)
